# Initial kernel scaffold; baseline (speedup 1.0000x reference)
import jax
import jax.numpy as jnp
from jax.experimental import pallas as pl


def kernel(indices, W):
    raise NotImplementedError("write your pallas kernel here")


# speedup vs baseline: 5.7383x; 5.7383x over previous
import functools

import jax
import jax.numpy as jnp
from jax import lax
from jax.experimental import pallas as pl
from jax.experimental.pallas import tpu as pltpu
from jax.experimental.pallas import tpu_sc as plsc

NUM_EMB = 1002
EMBED = 128
B, L = 4096, 200
N = B * L
NC, NS = 2, 16
NW = NC * NS
PER_W = N // NW
GATHER = 128
NJ = PER_W // GATHER


def _emb_body(idx_hbm, w_hbm, out_hbm, idx_v, rows_v, gsem, osem):
    cid = lax.axis_index("c")
    sid = lax.axis_index("s")
    wid = sid * NC + cid
    base = wid * PER_W

    pltpu.sync_copy(idx_hbm.at[wid], idx_v)

    def gather(j, buf):
        return pltpu.async_copy(w_hbm.at[idx_v.at[j]], rows_v.at[buf], gsem)

    def put(j, buf):
        return pltpu.async_copy(
            rows_v.at[buf], out_hbm.at[pl.ds(base + j * GATHER, GATHER)], osem
        )

    def wait_gather(buf):
        pltpu.make_async_copy(w_hbm.at[idx_v.at[0]], rows_v.at[buf], gsem).wait()

    def wait_put(j, buf):
        pltpu.make_async_copy(
            rows_v.at[buf], out_hbm.at[pl.ds(base + j * GATHER, GATHER)], osem
        ).wait()

    gather(0, 0)
    gather(1, 1)
    wait_gather(0)
    put(0, 0)

    @pl.loop(1, NJ - 1, step=2)
    def _steady(j0):
        for b in range(2):
            j = j0 + b
            buf = 1 - b
            wait_put(j - 1, 1 - buf)
            gather(j + 1, 1 - buf)
            wait_gather(buf)
            put(j, buf)

    wait_put(NJ - 2, 0)
    wait_gather(1)
    put(NJ - 1, 1)
    wait_put(NJ - 1, 1)


@functools.partial(jax.jit, static_argnames=())
def kernel(indices, W):
    idx = indices.reshape(NW, NJ, GATHER)
    mesh = plsc.VectorSubcoreMesh(
        core_axis_name="c", subcore_axis_name="s", num_cores=NC, num_subcores=NS
    )
    run = pl.kernel(
        _emb_body,
        out_type=jax.ShapeDtypeStruct((N, EMBED), jnp.float32),
        mesh=mesh,
        scratch_types=[
            pltpu.VMEM((NJ, GATHER), jnp.int32),
            pltpu.VMEM((2, GATHER, EMBED), jnp.float32),
            pltpu.SemaphoreType.DMA,
            pltpu.SemaphoreType.DMA,
        ],
    )
    out = run(idx, W)
    return out.reshape(B, L, EMBED)
